# baseline (speedup 1.0000x reference)
import functools

import jax
import jax.numpy as jnp
from jax import lax
from jax.experimental import pallas as pl
from jax.experimental.pallas import tpu as pltpu
from jax.experimental.pallas import tpu_sc as plsc

NUM_TABLES = 26
EMBED = 32
BATCH = 4096
OUT_D = NUM_TABLES * EMBED

_info = plsc.get_sparse_core_info()
_NC = _info.num_cores
_NS = _info.num_subcores
_NW = _NC * _NS
_BPW = BATCH // _NW

_mesh = plsc.VectorSubcoreMesh(core_axis_name="c", subcore_axis_name="s")


@functools.partial(
    pl.kernel,
    mesh=_mesh,
    out_type=jax.ShapeDtypeStruct((BATCH, OUT_D), jnp.float32),
    scratch_types=[
        pltpu.VMEM((NUM_TABLES, _BPW), jnp.int32),
        pltpu.VMEM((NUM_TABLES, _BPW, EMBED), jnp.float32),
        pltpu.SemaphoreType.DMA,
        pltpu.SemaphoreType.DMA,
    ],
    compiler_params=pltpu.CompilerParams(use_tc_tiling_on_sc=False),
)
def _embed_gather(feats_hbm, *rest):
    tables = rest[:NUM_TABLES]
    out_hbm = rest[NUM_TABLES]
    idx_v, rows_v, gsem, wsem = rest[NUM_TABLES + 1:]

    wid = lax.axis_index("s") * _NC + lax.axis_index("c")
    base = wid * _BPW

    pltpu.sync_copy(feats_hbm.at[:, pl.ds(base, _BPW)], idx_v)

    gathers = [
        pltpu.async_copy(tables[f].at[idx_v.at[f]], rows_v.at[f], gsem)
        for f in range(NUM_TABLES)
    ]
    writes = []
    for f in range(NUM_TABLES):
        gathers[f].wait()
        writes.append(
            pltpu.async_copy(
                rows_v.at[f],
                out_hbm.at[pl.ds(base, _BPW), pl.ds(f * EMBED, EMBED)],
                wsem,
            )
        )
    for w in writes:
        w.wait()


def kernel(
    feat_0, feat_1, feat_2, feat_3, feat_4, feat_5, feat_6, feat_7,
    feat_8, feat_9, feat_10, feat_11, feat_12, feat_13, feat_14, feat_15,
    feat_16, feat_17, feat_18, feat_19, feat_20, feat_21, feat_22, feat_23,
    feat_24, feat_25,
    W_0, W_1, W_2, W_3, W_4, W_5, W_6, W_7,
    W_8, W_9, W_10, W_11, W_12, W_13, W_14, W_15,
    W_16, W_17, W_18, W_19, W_20, W_21, W_22, W_23,
    W_24, W_25,
):
    feats = jnp.stack([
        feat_0, feat_1, feat_2, feat_3, feat_4, feat_5, feat_6, feat_7,
        feat_8, feat_9, feat_10, feat_11, feat_12, feat_13, feat_14, feat_15,
        feat_16, feat_17, feat_18, feat_19, feat_20, feat_21, feat_22,
        feat_23, feat_24, feat_25,
    ])
    return _embed_gather(
        feats,
        W_0, W_1, W_2, W_3, W_4, W_5, W_6, W_7,
        W_8, W_9, W_10, W_11, W_12, W_13, W_14, W_15,
        W_16, W_17, W_18, W_19, W_20, W_21, W_22, W_23,
        W_24, W_25,
    )
